# baseline (speedup 1.0000x reference)
import jax
import jax.numpy as jnp
from jax.experimental import pallas as pl
from jax.experimental.pallas import tpu as pltpu


def _build_hamilton(weight):
    a0, a1, a2, a3, a4, a5, a6, a7 = jnp.split(weight, 8, axis=1)
    rows = [
        [a0, a1, a2, a3, a4, a5, a6, a7],
        [a1, -a0, a3, -a2, a5, -a4, -a7, a6],
        [a2, -a3, -a0, a1, a6, a7, -a4, -a5],
        [a3, a2, -a1, -a0, a7, -a6, a5, -a4],
        [a4, -a5, -a6, -a7, -a0, a1, a2, a3],
        [a5, a4, -a7, a6, -a1, -a0, -a3, a2],
        [a6, a7, a4, -a5, -a2, a3, -a0, -a1],
        [a7, -a6, a5, a4, -a3, -a2, a1, -a0],
    ]
    return jnp.concatenate(
        [jnp.concatenate(r, axis=0) for r in rows], axis=1)


def _make_fused(n, out_f, bm):
    nblk = n // bm

    def fused(x_ref, h_ref, g_ref, b_ref, adj_ref, out_ref,
              sup_ref, y_ref, stat_ref, stage_ref, sem):
        i = pl.program_id(0)

        @pl.when(i == 0)
        def _init():
            sup_ref[...] = jnp.dot(x_ref[...], h_ref[...],
                                   preferred_element_type=jnp.float32)
            stat_ref[...] = jnp.zeros_like(stat_ref)

        y = jnp.dot(adj_ref[...], sup_ref[...],
                    preferred_element_type=jnp.float32)
        y_ref[pl.ds(i * bm, bm), :] = y
        stat_ref[0:1, :] += jnp.sum(y, axis=0, keepdims=True)
        stat_ref[1:2, :] += jnp.sum(y * y, axis=0, keepdims=True)

        @pl.when(i == nblk - 1)
        def _epilogue():
            mean = stat_ref[0:1, :] / n
            var = stat_ref[1:2, :] / n - mean * mean
            scale = jax.lax.rsqrt(var + 1e-5) * g_ref[...]
            shift = b_ref[...] - mean * scale

            def copy(j, slot):
                return pltpu.make_async_copy(
                    stage_ref.at[slot],
                    out_ref.at[pl.ds(j * bm, bm), :],
                    sem.at[slot])

            def body(j, _):
                slot = jax.lax.rem(j, 2)

                @pl.when(j >= 2)
                def _drain():
                    copy(j - 2, slot).wait()

                yb = y_ref[pl.ds(j * bm, bm), :]
                stage_ref[slot] = jnp.tanh(yb * scale + shift)
                copy(j, slot).start()
                return 0

            jax.lax.fori_loop(0, nblk, body, 0)
            copy(nblk - 2, jax.lax.rem(nblk - 2, 2)).wait()
            copy(nblk - 1, jax.lax.rem(nblk - 1, 2)).wait()

    return fused


def kernel(input, adj, weight, gamma, beta):
    n, in_f = input.shape
    out_f = weight.shape[1]
    hamilton = _build_hamilton(weight)

    bm = 400
    nblk = n // bm
    return pl.pallas_call(
        _make_fused(n, out_f, bm),
        grid=(nblk,),
        in_specs=[
            pl.BlockSpec((n, in_f), lambda i: (0, 0)),
            pl.BlockSpec((in_f, out_f), lambda i: (0, 0)),
            pl.BlockSpec((1, out_f), lambda i: (0, 0)),
            pl.BlockSpec((1, out_f), lambda i: (0, 0)),
            pl.BlockSpec((bm, n), lambda i: (i, 0)),
        ],
        out_specs=pl.BlockSpec(memory_space=pltpu.MemorySpace.HBM),
        out_shape=jax.ShapeDtypeStruct((n, out_f), jnp.float32),
        scratch_shapes=[
            pltpu.VMEM((n, out_f), jnp.float32),
            pltpu.VMEM((n, out_f), jnp.float32),
            pltpu.VMEM((8, out_f), jnp.float32),
            pltpu.VMEM((2, bm, out_f), jnp.float32),
            pltpu.SemaphoreType.DMA((2,)),
        ],
    )(input, hamilton, gamma.reshape(1, out_f), beta.reshape(1, out_f), adj)


# speedup vs baseline: 1.0448x; 1.0448x over previous
import jax
import jax.numpy as jnp
from jax.experimental import pallas as pl
from jax.experimental.pallas import tpu as pltpu


def _build_hamilton(weight):
    a0, a1, a2, a3, a4, a5, a6, a7 = jnp.split(weight, 8, axis=1)
    rows = [
        [a0, a1, a2, a3, a4, a5, a6, a7],
        [a1, -a0, a3, -a2, a5, -a4, -a7, a6],
        [a2, -a3, -a0, a1, a6, a7, -a4, -a5],
        [a3, a2, -a1, -a0, a7, -a6, a5, -a4],
        [a4, -a5, -a6, -a7, -a0, a1, a2, a3],
        [a5, a4, -a7, a6, -a1, -a0, -a3, a2],
        [a6, a7, a4, -a5, -a2, a3, -a0, -a1],
        [a7, -a6, a5, a4, -a3, -a2, a1, -a0],
    ]
    return jnp.concatenate(
        [jnp.concatenate(r, axis=0) for r in rows], axis=1)


def _make_fused(n, out_f, bm):
    nblk = n // bm

    def fused(x_ref, h_ref, g_ref, b_ref, adj_ref, out_ref,
              sup_ref, stat_ref):
        i = pl.program_id(0)

        @pl.when(i == 0)
        def _init():
            sup_ref[...] = jnp.dot(x_ref[...], h_ref[...],
                                   preferred_element_type=jnp.float32)
            stat_ref[...] = jnp.zeros_like(stat_ref)

        y = jnp.dot(adj_ref[...], sup_ref[...],
                    preferred_element_type=jnp.float32)
        out_ref[pl.ds(i * bm, bm), :] = y
        stat_ref[0:1, :] += jnp.sum(y, axis=0, keepdims=True)
        stat_ref[1:2, :] += jnp.sum(y * y, axis=0, keepdims=True)

        @pl.when(i == nblk - 1)
        def _epilogue():
            mean = stat_ref[0:1, :] / n
            var = stat_ref[1:2, :] / n - mean * mean
            scale = jax.lax.rsqrt(var + 1e-5) * g_ref[...]
            shift = b_ref[...] - mean * scale

            def body(j, _):
                yb = out_ref[pl.ds(j * bm, bm), :]
                out_ref[pl.ds(j * bm, bm), :] = jnp.tanh(yb * scale + shift)
                return 0

            jax.lax.fori_loop(0, nblk, body, 0)

    return fused


def kernel(input, adj, weight, gamma, beta):
    n, in_f = input.shape
    out_f = weight.shape[1]
    hamilton = _build_hamilton(weight)

    bm = 400
    nblk = n // bm
    return pl.pallas_call(
        _make_fused(n, out_f, bm),
        grid=(nblk,),
        in_specs=[
            pl.BlockSpec((n, in_f), lambda i: (0, 0)),
            pl.BlockSpec((in_f, out_f), lambda i: (0, 0)),
            pl.BlockSpec((1, out_f), lambda i: (0, 0)),
            pl.BlockSpec((1, out_f), lambda i: (0, 0)),
            pl.BlockSpec((bm, n), lambda i: (i, 0)),
        ],
        out_specs=pl.BlockSpec((n, out_f), lambda i: (0, 0)),
        out_shape=jax.ShapeDtypeStruct((n, out_f), jnp.float32),
        scratch_shapes=[
            pltpu.VMEM((n, out_f), jnp.float32),
            pltpu.VMEM((8, out_f), jnp.float32),
        ],
    )(input, hamilton, gamma.reshape(1, out_f), beta.reshape(1, out_f), adj)


# speedup vs baseline: 1.0506x; 1.0056x over previous
import jax
import jax.numpy as jnp
from jax.experimental import pallas as pl
from jax.experimental.pallas import tpu as pltpu


def _build_hamilton(weight):
    a0, a1, a2, a3, a4, a5, a6, a7 = jnp.split(weight, 8, axis=1)
    rows = [
        [a0, a1, a2, a3, a4, a5, a6, a7],
        [a1, -a0, a3, -a2, a5, -a4, -a7, a6],
        [a2, -a3, -a0, a1, a6, a7, -a4, -a5],
        [a3, a2, -a1, -a0, a7, -a6, a5, -a4],
        [a4, -a5, -a6, -a7, -a0, a1, a2, a3],
        [a5, a4, -a7, a6, -a1, -a0, -a3, a2],
        [a6, a7, a4, -a5, -a2, a3, -a0, -a1],
        [a7, -a6, a5, a4, -a3, -a2, a1, -a0],
    ]
    return jnp.concatenate(
        [jnp.concatenate(r, axis=0) for r in rows], axis=1)


def _make_fused(n, out_f, bm):
    nblk = n // bm

    def fused(x_ref, h_ref, g_ref, b_ref, adj_ref, out_ref,
              sup_ref, stat_ref):
        i = pl.program_id(0)

        @pl.when(i == 0)
        def _init():
            sup_ref[...] = jnp.dot(x_ref[...], h_ref[...],
                                   preferred_element_type=jnp.float32)
            stat_ref[...] = jnp.zeros_like(stat_ref)

        y = jnp.dot(adj_ref[...], sup_ref[...],
                    preferred_element_type=jnp.float32)
        out_ref[pl.ds(i * bm, bm), :] = y
        stat_ref[0:1, :] += jnp.sum(y, axis=0, keepdims=True)
        stat_ref[1:2, :] += jnp.sum(y * y, axis=0, keepdims=True)

        @pl.when(i == nblk - 1)
        def _epilogue():
            mean = stat_ref[0:1, :] / n
            var = stat_ref[1:2, :] / n - mean * mean
            scale = jax.lax.rsqrt(var + 1e-5) * g_ref[...]
            shift = b_ref[...] - mean * scale

            for j in range(nblk):
                yb = out_ref[j * bm:(j + 1) * bm, :]
                out_ref[j * bm:(j + 1) * bm, :] = jnp.tanh(yb * scale + shift)

    return fused


def kernel(input, adj, weight, gamma, beta):
    n, in_f = input.shape
    out_f = weight.shape[1]
    hamilton = _build_hamilton(weight)

    bm = 400
    nblk = n // bm
    return pl.pallas_call(
        _make_fused(n, out_f, bm),
        grid=(nblk,),
        in_specs=[
            pl.BlockSpec((n, in_f), lambda i: (0, 0)),
            pl.BlockSpec((in_f, out_f), lambda i: (0, 0)),
            pl.BlockSpec((1, out_f), lambda i: (0, 0)),
            pl.BlockSpec((1, out_f), lambda i: (0, 0)),
            pl.BlockSpec((bm, n), lambda i: (i, 0)),
        ],
        out_specs=pl.BlockSpec((n, out_f), lambda i: (0, 0)),
        out_shape=jax.ShapeDtypeStruct((n, out_f), jnp.float32),
        scratch_shapes=[
            pltpu.VMEM((n, out_f), jnp.float32),
            pltpu.VMEM((8, out_f), jnp.float32),
        ],
    )(input, hamilton, gamma.reshape(1, out_f), beta.reshape(1, out_f), adj)
